# baseline (speedup 1.0000x reference)
import functools

import jax
import jax.numpy as jnp
from jax import lax
from jax.experimental import pallas as pl
from jax.experimental.pallas import tpu as pltpu
from jax.experimental.pallas import tpu_sc as plsc

N = 10000
E = 320000
F = 128
NC = 2
NS = 16
NW = NC * NS
EPW = E // NW
CHUNKP = 128
NH = 4
NCHP = 20
EPWP = NH * NCHP * CHUNKP
NGRP = (NCHP * CHUNKP) // 16
GC = 64
CLEN = NCHP * CHUNKP + GC
NA = N + 16
DUMMY = N
ROWS_PER_SUB = 624
B = 5000
BPAD = 5120
BPW = BPAD // NW

_mesh = plsc.VectorSubcoreMesh(
    core_axis_name="c", subcore_axis_name="s", num_cores=NC, num_subcores=NS
)


@functools.partial(
    pl.kernel,
    out_type=(
        jax.ShapeDtypeStruct((NC, NA), jnp.float32),
        jax.ShapeDtypeStruct((NC, NA), jnp.float32),
    ),
    mesh=_mesh,
    scratch_types=[
        pltpu.VMEM((NCHP, CHUNKP), jnp.int32),
        pltpu.VMEM((2, BPW // 2), jnp.int32),
        pltpu.VMEM((CHUNKP,), jnp.float32),
        pltpu.VMEM_SHARED((NA,), jnp.float32),
        pltpu.VMEM_SHARED((NA,), jnp.float32),
    ],
)
def _deg_sc(dst_hbm, pid_hbm, zeros_hbm, cnt_hbm, pmk_hbm,
            dst_v, pid_v, ones_v, deg_sh, mask_sh):
    cid = lax.axis_index("c")
    sid = lax.axis_index("s")
    wid = cid * NS + sid
    for j in range(CHUNKP // 16):
        ones_v[pl.ds(j * 16, 16)] = jnp.ones((16,), jnp.float32)

    @pl.when(sid == 0)
    def _():
        pltpu.sync_copy(zeros_hbm, deg_sh)

    @pl.when(sid == 1)
    def _():
        pltpu.sync_copy(zeros_hbm, mask_sh)

    plsc.subcore_barrier()
    pltpu.sync_copy(pid_hbm.at[wid], pid_v)
    for j in range(2):
        pltpu.sync_copy(
            ones_v.at[pl.ds(0, BPW // 2)], mask_sh.at[pid_v.at[j]], add=True
        )
    for h in range(NH):
        pltpu.sync_copy(dst_hbm.at[wid, h], dst_v)

        @pl.loop(0, NCHP)
        def _(j):
            pltpu.sync_copy(ones_v, deg_sh.at[dst_v.at[j]], add=True)

    plsc.subcore_barrier()

    @pl.when(sid == 0)
    def _():
        pltpu.sync_copy(deg_sh, cnt_hbm.at[cid])

    @pl.when(sid == 1)
    def _():
        pltpu.sync_copy(mask_sh, pmk_hbm.at[cid])


def _gmul_body(x_ref, w_ref, cnt_ref, pmk_ref, g_ref, nf_ref):
    cnt = cnt_ref[...]
    deg = cnt[0, :N] + cnt[1, :N] + 1.0
    dinv = lax.rsqrt(deg)[:, None]
    h = jnp.dot(x_ref[...], w_ref[...], preferred_element_type=jnp.float32)
    g_ref[...] = h * dinv
    pmk = pmk_ref[...]
    need = jnp.where(pmk[0, :] + pmk[1, :] > 0.0, 1.0, 0.0)
    nf_ref[...] = need[:, None]


def _gmul(x, w, cnt, pmk):
    return pl.pallas_call(
        _gmul_body,
        out_shape=(
            jax.ShapeDtypeStruct((N, F), jnp.float32),
            jax.ShapeDtypeStruct((NA, 1), jnp.float32),
        ),
    )(x, w, cnt, pmk)


@functools.partial(
    pl.kernel,
    out_type=jax.ShapeDtypeStruct((NC, N, F), jnp.float32),
    mesh=_mesh,
    scratch_types=[
        pltpu.VMEM((NCHP, CHUNKP), jnp.int32),
        pltpu.VMEM((NCHP, CHUNKP), jnp.int32),
        pltpu.VMEM((NCHP, CHUNKP), jnp.float32),
        pltpu.VMEM((CLEN // GC, GC), jnp.int32),
        pltpu.VMEM((CLEN // GC, GC), jnp.int32),
        pltpu.VMEM((GC, F), jnp.float32),
        pltpu.VMEM((GC, F), jnp.float32),
        pltpu.VMEM_SHARED((NA, F), jnp.float32),
        pltpu.SemaphoreType.DMA,
        pltpu.SemaphoreType.DMA,
        pltpu.SemaphoreType.DMA,
    ],
    compiler_params=pltpu.CompilerParams(needs_layout_passes=False),
)
def _scat_sc(src_hbm, dst_hbm, g_hbm, nf_hbm, zeros_hbm, out_hbm,
             src_v, dst_v, mv_v, csrc_v, cdst_v, rows0_v, rows1_v,
             acc_sh, sem0, sem1, msem):
    cid = lax.axis_index("c")
    sid = lax.axis_index("s")
    wid = cid * NS + sid
    r0 = sid * ROWS_PER_SUB
    pltpu.sync_copy(
        zeros_hbm.at[pl.ds(r0, ROWS_PER_SUB)],
        acc_sh.at[pl.ds(r0, ROWS_PER_SUB)],
    )

    @pl.when(sid == NS - 1)
    def _():
        rem = NS * ROWS_PER_SUB
        pltpu.sync_copy(
            zeros_hbm.at[pl.ds(rem, N - rem)],
            acc_sh.at[pl.ds(rem, N - rem)],
        )

    plsc.subcore_barrier()

    def gstart(c, buf, sem):
        pltpu.make_async_copy(g_hbm.at[csrc_v.at[c]], buf, sem).start()

    def gwait(buf, sem):
        pltpu.make_async_copy(g_hbm.at[csrc_v.at[0]], buf, sem).wait()

    for h in range(NH):
        pltpu.sync_copy(src_hbm.at[wid, h], src_v)
        pltpu.sync_copy(dst_hbm.at[wid, h], dst_v)
        @pl.loop(0, NCHP)
        def _(j):
            pltpu.make_async_copy(
                nf_hbm.at[dst_v.at[j]], mv_v.at[j], msem
            ).start()

        @pl.loop(0, NCHP)
        def _(j):
            pltpu.make_async_copy(
                nf_hbm.at[dst_v.at[0]], mv_v.at[0], msem
            ).wait()

        def _compress(grp, cursor):
            row = grp // 8
            col = 16 * lax.rem(grp, 8)
            m = mv_v[row, pl.ds(col, 16)] > 0.5
            s = src_v[row, pl.ds(col, 16)]
            d = dst_v[row, pl.ds(col, 16)]
            mi = m.astype(jnp.int32)
            pos = cursor + plsc.cumsum(mi) - mi
            pr = pos // GC
            pc = lax.rem(pos, GC)
            plsc.store_scatter(csrc_v, [pr, pc], s, mask=m)
            plsc.store_scatter(cdst_v, [pr, pc], d, mask=m)
            return cursor + jnp.sum(mi)

        cursor = pl.loop(0, NGRP, init_carry=jnp.int32(0))(_compress)

        npad = lax.rem(GC - lax.rem(cursor, GC), GC)
        iota16 = lax.iota(jnp.int32, 16)
        for k in range(GC // 16):
            @pl.when(16 * k < npad)
            def _():
                ppos = cursor + 16 * k + iota16
                plsc.store_scatter(csrc_v, [ppos // GC, lax.rem(ppos, GC)],
                                   jnp.zeros((16,), jnp.int32))
                plsc.store_scatter(cdst_v, [ppos // GC, lax.rem(ppos, GC)],
                                   jnp.full((16,), DUMMY, jnp.int32))

        nc = (cursor + npad) // GC

        @pl.when(nc > 0)
        def _():
            gstart(0, rows0_v, sem0)

        @pl.loop(0, nc)
        def _(c):
            par = lax.rem(c, 2)
            more = c + 1 < nc

            @pl.when(jnp.logical_and(more, par == 0))
            def _():
                gstart(c + 1, rows1_v, sem1)

            @pl.when(jnp.logical_and(more, par == 1))
            def _():
                gstart(c + 1, rows0_v, sem0)

            @pl.when(par == 0)
            def _():
                gwait(rows0_v, sem0)
                pltpu.sync_copy(rows0_v, acc_sh.at[cdst_v.at[c]], add=True)

            @pl.when(par == 1)
            def _():
                gwait(rows1_v, sem1)
                pltpu.sync_copy(rows1_v, acc_sh.at[cdst_v.at[c]], add=True)

    plsc.subcore_barrier()
    pltpu.sync_copy(
        acc_sh.at[pl.ds(r0, ROWS_PER_SUB)],
        out_hbm.at[cid].at[pl.ds(r0, ROWS_PER_SUB)],
    )

    @pl.when(sid == NS - 1)
    def _():
        rem = NS * ROWS_PER_SUB
        pltpu.sync_copy(
            acc_sh.at[pl.ds(rem, N - rem)],
            out_hbm.at[cid].at[pl.ds(rem, N - rem)],
        )


def _head_body(s_ref, g_ref, cnt_ref, bgc_ref, w00_ref, b00_ref, w10_ref,
               b10_ref, w01_ref, b01_ref, w11_ref, b11_ref, y0_ref, y1_ref):
    cnt = cnt_ref[...]
    deg = cnt[0, :N] + cnt[1, :N] + 1.0
    dinv = lax.rsqrt(deg)[:, None]
    s = s_ref[0] + s_ref[1] + g_ref[...]
    dist = jnp.maximum(s * dinv + bgc_ref[...][None, :], 0.0)
    y00 = jnp.maximum(
        jnp.dot(dist, w00_ref[...], preferred_element_type=jnp.float32)
        + b00_ref[...][None, :], 0.0)
    y10 = jnp.maximum(
        jnp.dot(dist, w10_ref[...], preferred_element_type=jnp.float32)
        + b10_ref[...][None, :], 0.0)
    z0 = jnp.dot(y00, w01_ref[...], preferred_element_type=jnp.float32)
    z1 = jnp.dot(y10, w11_ref[...], preferred_element_type=jnp.float32)
    y0_ref[...] = jax.nn.sigmoid(z0 + b01_ref[...][None, :])
    y1_ref[...] = jax.nn.sigmoid(z1 + b11_ref[...][None, :])


def _head(s, g, cnt, bgc, w00, b00, w10, b10, w01, b01, w11, b11):
    return pl.pallas_call(
        _head_body,
        out_shape=(
            jax.ShapeDtypeStruct((N, 1), jnp.float32),
            jax.ShapeDtypeStruct((N, 1), jnp.float32),
        ),
    )(s, g, cnt, bgc, w00, b00, w10, b10, w01, b01, w11, b11)


@functools.partial(
    pl.kernel,
    out_type=(
        jax.ShapeDtypeStruct((BPAD,), jnp.float32),
        jax.ShapeDtypeStruct((BPAD,), jnp.float32),
        jax.ShapeDtypeStruct((BPAD,), jnp.float32),
    ),
    mesh=_mesh,
    scratch_types=[
        pltpu.VMEM((BPW,), jnp.int32),
        pltpu.VMEM((BPW,), jnp.int32),
        pltpu.VMEM((BPW,), jnp.float32),
        pltpu.VMEM((BPW,), jnp.float32),
        pltpu.VMEM((BPW,), jnp.float32),
        pltpu.SemaphoreType.DMA,
    ],
)
def _pick_sc(y0_hbm, y1_hbm, pid_hbm, t_hbm, y_out, y1_out, y0_out,
             pid_v, t_v, g0_v, g1_v, oy_v, sem):
    cid = lax.axis_index("c")
    sid = lax.axis_index("s")
    wid = cid * NS + sid
    base = wid * BPW
    pltpu.sync_copy(pid_hbm.at[pl.ds(base, BPW)], pid_v)
    pltpu.sync_copy(t_hbm.at[pl.ds(base, BPW)], t_v)
    pltpu.async_copy(y0_hbm.at[pid_v], g0_v, sem).wait()
    pltpu.async_copy(y1_hbm.at[pid_v], g1_v, sem).wait()
    for i in range(BPW // 16):
        sl = pl.ds(i * 16, 16)
        oy_v[sl] = jnp.where(t_v[sl] > 0, g1_v[sl], g0_v[sl])
    pltpu.sync_copy(oy_v, y_out.at[pl.ds(base, BPW)])
    pltpu.sync_copy(g1_v, y1_out.at[pl.ds(base, BPW)])
    pltpu.sync_copy(g0_v, y0_out.at[pl.ds(base, BPW)])


def kernel(x, edge_index, patient_ids, treatment, W_gc, b_gc, W_t00, b_t00,
           W_t10, b_t10, W_t01, b_t01, W_t11, b_t11):
    srcw = edge_index[0].astype(jnp.int32).reshape(NW, EPW)
    dstw = edge_index[1].astype(jnp.int32).reshape(NW, EPW)
    pad = EPWP - EPW
    srcp = jnp.pad(srcw, ((0, 0), (0, pad))).reshape(NW, NH, NCHP, CHUNKP)
    dstp = jnp.pad(dstw, ((0, 0), (0, pad)),
                   constant_values=DUMMY).reshape(NW, NH, NCHP, CHUNKP)
    zeros_n = jnp.zeros((NA,), jnp.float32)
    zeros_nf = jnp.zeros((N, F), jnp.float32)

    pad_i = jnp.zeros((BPAD - B,), jnp.int32)
    pid = jnp.concatenate([patient_ids.astype(jnp.int32), pad_i])
    tre = jnp.concatenate([treatment.astype(jnp.int32), pad_i])

    cnt, pmk = _deg_sc(dstp, pid.reshape(NW, 2, BPW // 2), zeros_n)
    g, needf = _gmul(x, W_gc, cnt, pmk)
    s = _scat_sc(srcp, dstp, g, needf.reshape(NA), zeros_nf)
    y0, y1 = _head(s, g, cnt, b_gc, W_t00, b_t00, W_t10, b_t10,
                   W_t01, b_t01, W_t11, b_t11)
    y0 = y0.reshape(N)
    y1 = y1.reshape(N)

    y, y1p, y0p = _pick_sc(y0, y1, pid, tre)
    return y[:B], y1p[:B], y0p[:B]
